# baseline (speedup 1.0000x reference)
import functools

import jax
import jax.numpy as jnp
from jax import lax
from jax.experimental import pallas as pl
from jax.experimental.pallas import tpu as pltpu
from jax.experimental.pallas import tpu_sc as plsc

N = 10000
NP = 10240
D = 128
DH = D // 2
E = 320000
NC = 2
NS = 16
NW = NC * NS
K = 256
NBUF = 4
SLAG = 2
NCH = 80
NPASS = 4
PCH = NCH // NPASS
EPAD = NS * NCH * K
ECH = EPAD // K
DCH = ECH // NW
RPT = NP // NS
L = 16
BM = 256
GB = NP // BM


def _bcast_lane(v, j):
    return jnp.take_along_axis(
        v, jnp.full((L,), j, jnp.int32), axis=0, mode="promise_in_bounds"
    )


def _sc_deg_body(col_hbm, ew_hbm, z1d_hbm,
                 deg2_hbm,
                 deg_sp, bufB, bufC):
    c = lax.axis_index("c")
    s = lax.axis_index("s")
    w = s * NC + c

    pltpu.sync_copy(z1d_hbm, deg_sp.at[pl.ds(s * RPT, RPT)])
    plsc.subcore_barrier()

    base = w * DCH
    pltpu.sync_copy(ew_hbm.at[pl.ds(base, DCH)], bufB)
    pltpu.sync_copy(col_hbm.at[pl.ds(base, DCH)], bufC)

    @pl.loop(0, DCH)
    def _dgi(i):
        pltpu.sync_copy(bufB.at[i], deg_sp.at[bufC.at[i]], add=True)

    plsc.subcore_barrier()
    pltpu.sync_copy(deg_sp.at[pl.ds(s * RPT, RPT)],
                    deg2_hbm.at[c, pl.ds(s * RPT, RPT)])


@functools.lru_cache(maxsize=None)
def _sc_deg():
    mesh = plsc.VectorSubcoreMesh(core_axis_name="c", subcore_axis_name="s")
    return pl.kernel(
        _sc_deg_body,
        out_type=[jax.ShapeDtypeStruct((NC, NP), jnp.float32)],
        mesh=mesh,
        compiler_params=pltpu.CompilerParams(
            needs_layout_passes=False, use_tc_tiling_on_sc=False),
        scratch_types=[
            pltpu.VMEM_SHARED((NP,), jnp.float32),
            pltpu.VMEM((DCH, K), jnp.float32),
            pltpu.VMEM((DCH, K), jnp.int32),
        ],
    )


def _pipeline_pass(ycat_hbm, acc_sp, bufA, bufB, bufC, rows, sgs, ss):
    def g_start(i, b):
        pltpu.async_copy(ycat_hbm.at[bufA.at[i]], rows[b], sgs[b])

    def g_wait(i, b):
        pltpu.make_async_copy(ycat_hbm.at[bufA.at[i]], rows[b], sgs[b]).wait()

    def s_wait(i, b):
        pltpu.make_async_copy(rows[b], acc_sp.at[bufC.at[i]], ss).wait()

    for b in range(NBUF):
        g_start(b, b)

    @pl.loop(0, PCH // NBUF)
    def _blk(ii):
        for b in range(NBUF):
            i = ii * NBUF + b
            g_wait(i, b)

            @pl.loop(0, K // L)
            def _grp(g):
                g16 = pl.multiple_of(g * L, L)
                nv = bufB[i, pl.ds(g16, L)]
                for j in range(L):
                    nb = _bcast_lane(nv, j)
                    e = g16 + j
                    for d in range(DH // L):
                        sld = pl.ds(d * L, L)
                        rows[b][e, sld] = rows[b][e, sld] * nb

            bp = (b - SLAG) % NBUF

            @pl.when(i >= SLAG)
            def _():
                s_wait(i - SLAG, bp)

                @pl.when(i + NBUF - SLAG < PCH)
                def _():
                    g_start(i + NBUF - SLAG, bp)

            pltpu.async_copy(rows[b], acc_sp.at[bufC.at[i]], ss, add=True)

    for q in range(SLAG):
        s_wait(PCH - SLAG + q, (PCH - SLAG + q) % NBUF)


def _sc_layer_body(row_hbm, col_hbm, ew_hbm, ycat_hbm, z2d_hbm,
                   acc_hbm,
                   acc_sp, bufA, bufB, bufC, rows0, rows1, rows2, rows3,
                   sg0, sg1, sg2, sg3, ss0):
    c = lax.axis_index("c")
    s = lax.axis_index("s")
    rows = (rows0, rows1, rows2, rows3)
    sgs = (sg0, sg1, sg2, sg3)

    pltpu.sync_copy(z2d_hbm, acc_sp.at[pl.ds(s * RPT, RPT)])
    plsc.subcore_barrier()

    cnp = c * NP

    @pl.loop(0, NPASS)
    def _mp(p):
        base = s * NCH + p * PCH
        pltpu.sync_copy(row_hbm.at[pl.ds(base, PCH)], bufA)
        pltpu.sync_copy(ew_hbm.at[pl.ds(base, PCH)], bufB)
        pltpu.sync_copy(col_hbm.at[pl.ds(base, PCH)], bufC)

        @pl.loop(0, PCH)
        def _ofs(i):
            for g in range(K // L):
                sl = pl.ds(g * L, L)
                bufA[i, sl] = bufA[i, sl] + cnp

        _pipeline_pass(ycat_hbm, acc_sp, bufA, bufB, bufC, rows, sgs, ss0)

    plsc.subcore_barrier()
    pltpu.sync_copy(acc_sp.at[pl.ds(s * RPT, RPT)],
                    acc_hbm.at[pl.ds(c * NP + s * RPT, RPT)])


@functools.lru_cache(maxsize=None)
def _sc_layer():
    mesh = plsc.VectorSubcoreMesh(core_axis_name="c", subcore_axis_name="s")
    return pl.kernel(
        _sc_layer_body,
        out_type=[jax.ShapeDtypeStruct((NC * NP, DH), jnp.float32)],
        mesh=mesh,
        compiler_params=pltpu.CompilerParams(
            needs_layout_passes=False, use_tc_tiling_on_sc=False),
        scratch_types=[
            pltpu.VMEM_SHARED((NP, DH), jnp.float32),
            pltpu.VMEM((PCH, K), jnp.int32),
            pltpu.VMEM((PCH, K), jnp.float32),
            pltpu.VMEM((PCH, K), jnp.int32),
            pltpu.VMEM((K, DH), jnp.float32),
            pltpu.VMEM((K, DH), jnp.float32),
            pltpu.VMEM((K, DH), jnp.float32),
            pltpu.VMEM((K, DH), jnp.float32),
        ] + [pltpu.SemaphoreType.DMA] * 5,
    )


def _dis_of(deg_lo_ref, deg_hi_ref):
    return lax.rsqrt(deg_lo_ref[...] + deg_hi_ref[...] + 1.0)


def _m1_body(x_ref, w_ref, dlo_ref, dhi_ref, o_ref):
    d = _dis_of(dlo_ref, dhi_ref)
    o_ref[...] = d * jnp.dot(x_ref[...], w_ref[0],
                             preferred_element_type=jnp.float32)


def _m2_body(p_lo, p_hi, y_lo, y_hi, dlo_ref, dhi_ref, b_ref, w_ref, o_ref):
    d = _dis_of(dlo_ref, dhi_ref)
    scat = jnp.concatenate(
        [p_lo[...] + y_lo[...], p_hi[...] + y_hi[...]], axis=1)
    h = jnp.maximum(d * scat + b_ref[...], 0.0)
    o_ref[...] = d * jnp.dot(h, w_ref[0], preferred_element_type=jnp.float32)


def _m3_body(p_lo, p_hi, y_lo, y_hi, dlo_ref, dhi_ref, b_ref, w_ref, b3_ref,
             o_ref):
    d = _dis_of(dlo_ref, dhi_ref)
    scat = jnp.concatenate(
        [p_lo[...] + y_lo[...], p_hi[...] + y_hi[...]], axis=1)
    h = jnp.maximum(d * scat + b_ref[...], 0.0)
    o_ref[...] = jnp.tanh(
        jnp.dot(h, w_ref[...], preferred_element_type=jnp.float32)
        + b3_ref[...]
    )


_lo_spec = pl.BlockSpec((BM, DH), lambda i, c: (i, 0))
_hi_spec = pl.BlockSpec((BM, DH), lambda i, c: (GB + i, 0))
_whalf_spec = pl.BlockSpec((1, D, DH), lambda i, c: (c, 0, 0))
_ohalf_spec = pl.BlockSpec((BM, DH), lambda i, c: (c * GB + i, 0))
_dis_spec = pl.BlockSpec((BM, 1), lambda i, c: (i, 0))
_b_spec = pl.BlockSpec((1, D), lambda i, c: (0, 0))
_cat_sds = jax.ShapeDtypeStruct((NC * NP, DH), jnp.float32)


def _m1(x, w, dlo, dhi):
    return pl.pallas_call(
        _m1_body,
        grid=(GB, NC),
        in_specs=[pl.BlockSpec((BM, D), lambda i, c: (i, 0)), _whalf_spec,
                  _dis_spec, _dis_spec],
        out_specs=_ohalf_spec,
        out_shape=_cat_sds,
    )(x, w, dlo, dhi)


def _m2(acc, ycat, dlo, dhi, b, w):
    return pl.pallas_call(
        _m2_body,
        grid=(GB, NC),
        in_specs=[_lo_spec, _hi_spec, _lo_spec, _hi_spec, _dis_spec,
                  _dis_spec, _b_spec, _whalf_spec],
        out_specs=_ohalf_spec,
        out_shape=_cat_sds,
    )(acc, acc, ycat, ycat, dlo, dhi, b, w)


def _m3(acc, ycat, dlo, dhi, b, w, b3):
    return pl.pallas_call(
        _m3_body,
        grid=(GB,),
        in_specs=[
            pl.BlockSpec((BM, DH), lambda i: (i, 0)),
            pl.BlockSpec((BM, DH), lambda i: (GB + i, 0)),
            pl.BlockSpec((BM, DH), lambda i: (i, 0)),
            pl.BlockSpec((BM, DH), lambda i: (GB + i, 0)),
            pl.BlockSpec((BM, 1), lambda i: (i, 0)),
            pl.BlockSpec((BM, 1), lambda i: (i, 0)),
            pl.BlockSpec((1, D), lambda i: (0, 0)),
            pl.BlockSpec((D, D), lambda i: (0, 0)),
            pl.BlockSpec((1, D), lambda i: (0, 0)),
        ],
        out_specs=pl.BlockSpec((BM, D), lambda i: (i, 0)),
        out_shape=jax.ShapeDtypeStruct((NP, D), jnp.float32),
    )(acc, acc, ycat, ycat, dlo, dhi, b, w, b3)


def _first(x):
    return x[0] if isinstance(x, (tuple, list)) else x


def kernel(x, edge_index, edge_weight, W1, b1, W2, b2, W3, b3):
    row = edge_index[0].astype(jnp.int32)
    col = edge_index[1].astype(jnp.int32)
    ew = edge_weight.astype(jnp.float32)

    pad = EPAD - E
    row_p = jnp.concatenate([row, jnp.zeros((pad,), jnp.int32)]).reshape(ECH, K)
    col_p = jnp.concatenate([col, jnp.full((pad,), N, jnp.int32)]).reshape(ECH, K)
    ew_p = jnp.concatenate([ew, jnp.zeros((pad,), jnp.float32)]).reshape(ECH, K)
    x_p = jnp.pad(x, ((0, NP - N), (0, 0)))
    z2d = jnp.zeros((RPT, DH), jnp.float32)
    z1d = jnp.zeros((RPT,), jnp.float32)
    w1h = W1.astype(jnp.float32).reshape(D, NC, DH).transpose(1, 0, 2)
    w2h = W2.astype(jnp.float32).reshape(D, NC, DH).transpose(1, 0, 2)

    deg2 = _first(_sc_deg()(col_p, ew_p, z1d))
    dlo = deg2[0][:, None]
    dhi = deg2[1][:, None]

    y1 = _m1(x_p, w1h, dlo, dhi)
    acc1 = _first(_sc_layer()(row_p, col_p, ew_p, y1, z2d))
    y2 = _m2(acc1, y1, dlo, dhi, b1.reshape(1, D), w2h)
    acc2 = _first(_sc_layer()(row_p, col_p, ew_p, y2, z2d))
    out = _m3(acc2, y2, dlo, dhi, b2.reshape(1, D), W3, b3.reshape(1, D))
    return out[:N]


# speedup vs baseline: 1.4413x; 1.4413x over previous
import functools

import jax
import jax.numpy as jnp
from jax import lax
from jax.experimental import pallas as pl
from jax.experimental.pallas import tpu as pltpu
from jax.experimental.pallas import tpu_sc as plsc

N = 10000
NP = 10240
D = 128
DH = D // 2
E = 320000
NC = 2
NS = 16
NW = NC * NS
K = 128
NBUF = 4
NCH = 160
NPASS = 2
PCH = NCH // NPASS
EPAD = NS * NCH * K
ECH = EPAD // K
RPT = NP // NS
L = 16
BM = 256
GB = NP // BM


def _bcast_lane(v, j):
    return jnp.take_along_axis(
        v, jnp.full((L,), j, jnp.int32), axis=0, mode="promise_in_bounds"
    )


def _sc_deg_body(col_hbm, ew_hbm, z1d_hbm,
                 deg2_hbm,
                 deg_sp, bufB, bufC):
    c = lax.axis_index("c")
    s = lax.axis_index("s")
    w = s * NC + c

    pltpu.sync_copy(z1d_hbm, deg_sp.at[pl.ds(s * RPT, RPT)])
    plsc.subcore_barrier()

    base = w * PCH
    pltpu.sync_copy(ew_hbm.at[pl.ds(base, PCH)], bufB)
    pltpu.sync_copy(col_hbm.at[pl.ds(base, PCH)], bufC)

    @pl.loop(0, PCH)
    def _dgi(i):
        pltpu.sync_copy(bufB.at[i], deg_sp.at[bufC.at[i]], add=True)

    plsc.subcore_barrier()
    pltpu.sync_copy(deg_sp.at[pl.ds(s * RPT, RPT)],
                    deg2_hbm.at[c, pl.ds(s * RPT, RPT)])


@functools.lru_cache(maxsize=None)
def _sc_deg():
    mesh = plsc.VectorSubcoreMesh(core_axis_name="c", subcore_axis_name="s")
    return pl.kernel(
        _sc_deg_body,
        out_type=[jax.ShapeDtypeStruct((NC, NP), jnp.float32)],
        mesh=mesh,
        compiler_params=pltpu.CompilerParams(
            needs_layout_passes=False, use_tc_tiling_on_sc=False),
        scratch_types=[
            pltpu.VMEM_SHARED((NP,), jnp.float32),
            pltpu.VMEM((PCH, K), jnp.float32),
            pltpu.VMEM((PCH, K), jnp.int32),
        ],
    )


def _pipeline_pass(ycat_hbm, acc_sp, bufA, bufB, bufC, rows, sgs, ss):
    def g_start(i, b):
        pltpu.async_copy(ycat_hbm.at[bufA.at[i]], rows[b], sgs[b])

    def g_wait(i, b):
        pltpu.make_async_copy(ycat_hbm.at[bufA.at[i]], rows[b], sgs[b]).wait()

    def s_wait(i, b):
        pltpu.make_async_copy(rows[b], acc_sp.at[bufC.at[i]], ss).wait()

    for b in range(NBUF):
        g_start(b, b)

    @pl.loop(0, PCH // NBUF)
    def _blk(ii):
        for b in range(NBUF):
            i = ii * NBUF + b
            g_wait(i, b)

            @pl.loop(0, K // L, unroll=2)
            def _grp(g):
                g16 = pl.multiple_of(g * L, L)
                nv = bufB[i, pl.ds(g16, L)]
                for j in range(L):
                    nb = _bcast_lane(nv, j)
                    e = g16 + j
                    for d in range(DH // L):
                        sld = pl.ds(d * L, L)
                        rows[b][e, sld] = rows[b][e, sld] * nb

            bp = (b - 1) % NBUF

            @pl.when(i >= 1)
            def _():
                s_wait(i - 1, bp)

                @pl.when(i + NBUF - 1 < PCH)
                def _():
                    g_start(i + NBUF - 1, bp)

            pltpu.async_copy(rows[b], acc_sp.at[bufC.at[i]], ss, add=True)

    s_wait(PCH - 1, (PCH - 1) % NBUF)


def _sc_layer_body(row_hbm, col_hbm, ew_hbm, ycat_hbm, z2d_hbm,
                   acc_hbm,
                   acc_sp, bufA, bufB, bufC, rows0, rows1, rows2, rows3,
                   sg0, sg1, sg2, sg3, ss0):
    c = lax.axis_index("c")
    s = lax.axis_index("s")
    rows = (rows0, rows1, rows2, rows3)
    sgs = (sg0, sg1, sg2, sg3)

    pltpu.sync_copy(z2d_hbm, acc_sp.at[pl.ds(s * RPT, RPT)])
    plsc.subcore_barrier()

    cnp = c * NP

    @pl.loop(0, NPASS)
    def _mp(p):
        base = s * NCH + p * PCH
        pltpu.sync_copy(row_hbm.at[pl.ds(base, PCH)], bufA)
        pltpu.sync_copy(ew_hbm.at[pl.ds(base, PCH)], bufB)
        pltpu.sync_copy(col_hbm.at[pl.ds(base, PCH)], bufC)

        @pl.loop(0, PCH)
        def _ofs(i):
            for g in range(K // L):
                sl = pl.ds(g * L, L)
                bufA[i, sl] = bufA[i, sl] + cnp

        _pipeline_pass(ycat_hbm, acc_sp, bufA, bufB, bufC, rows, sgs, ss0)

    plsc.subcore_barrier()
    pltpu.sync_copy(acc_sp.at[pl.ds(s * RPT, RPT)],
                    acc_hbm.at[pl.ds(c * NP + s * RPT, RPT)])


@functools.lru_cache(maxsize=None)
def _sc_layer():
    mesh = plsc.VectorSubcoreMesh(core_axis_name="c", subcore_axis_name="s")
    return pl.kernel(
        _sc_layer_body,
        out_type=[jax.ShapeDtypeStruct((NC * NP, DH), jnp.float32)],
        mesh=mesh,
        compiler_params=pltpu.CompilerParams(
            needs_layout_passes=False, use_tc_tiling_on_sc=False),
        scratch_types=[
            pltpu.VMEM_SHARED((NP, DH), jnp.float32),
            pltpu.VMEM((PCH, K), jnp.int32),
            pltpu.VMEM((PCH, K), jnp.float32),
            pltpu.VMEM((PCH, K), jnp.int32),
            pltpu.VMEM((K, DH), jnp.float32),
            pltpu.VMEM((K, DH), jnp.float32),
            pltpu.VMEM((K, DH), jnp.float32),
            pltpu.VMEM((K, DH), jnp.float32),
        ] + [pltpu.SemaphoreType.DMA] * 5,
    )


def _dis_of(deg_lo_ref, deg_hi_ref):
    return lax.rsqrt(deg_lo_ref[...] + deg_hi_ref[...] + 1.0)


def _m1_body(x_ref, w_ref, dlo_ref, dhi_ref, o_ref):
    d = _dis_of(dlo_ref, dhi_ref)
    o_ref[...] = d * jnp.dot(x_ref[...], w_ref[0],
                             preferred_element_type=jnp.float32)


def _m2_body(p_lo, p_hi, y_lo, y_hi, dlo_ref, dhi_ref, b_ref, w_ref, o_ref):
    d = _dis_of(dlo_ref, dhi_ref)
    scat = jnp.concatenate(
        [p_lo[...] + y_lo[...], p_hi[...] + y_hi[...]], axis=1)
    h = jnp.maximum(d * scat + b_ref[...], 0.0)
    o_ref[...] = d * jnp.dot(h, w_ref[0], preferred_element_type=jnp.float32)


def _m3_body(p_lo, p_hi, y_lo, y_hi, dlo_ref, dhi_ref, b_ref, w_ref, b3_ref,
             o_ref):
    d = _dis_of(dlo_ref, dhi_ref)
    scat = jnp.concatenate(
        [p_lo[...] + y_lo[...], p_hi[...] + y_hi[...]], axis=1)
    h = jnp.maximum(d * scat + b_ref[...], 0.0)
    o_ref[...] = jnp.tanh(
        jnp.dot(h, w_ref[...], preferred_element_type=jnp.float32)
        + b3_ref[...]
    )


_lo_spec = pl.BlockSpec((BM, DH), lambda i, c: (i, 0))
_hi_spec = pl.BlockSpec((BM, DH), lambda i, c: (GB + i, 0))
_whalf_spec = pl.BlockSpec((1, D, DH), lambda i, c: (c, 0, 0))
_ohalf_spec = pl.BlockSpec((BM, DH), lambda i, c: (c * GB + i, 0))
_dis_spec = pl.BlockSpec((BM, 1), lambda i, c: (i, 0))
_b_spec = pl.BlockSpec((1, D), lambda i, c: (0, 0))
_cat_sds = jax.ShapeDtypeStruct((NC * NP, DH), jnp.float32)


def _m1(x, w, dlo, dhi):
    return pl.pallas_call(
        _m1_body,
        grid=(GB, NC),
        in_specs=[pl.BlockSpec((BM, D), lambda i, c: (i, 0)), _whalf_spec,
                  _dis_spec, _dis_spec],
        out_specs=_ohalf_spec,
        out_shape=_cat_sds,
    )(x, w, dlo, dhi)


def _m2(acc, ycat, dlo, dhi, b, w):
    return pl.pallas_call(
        _m2_body,
        grid=(GB, NC),
        in_specs=[_lo_spec, _hi_spec, _lo_spec, _hi_spec, _dis_spec,
                  _dis_spec, _b_spec, _whalf_spec],
        out_specs=_ohalf_spec,
        out_shape=_cat_sds,
    )(acc, acc, ycat, ycat, dlo, dhi, b, w)


def _m3(acc, ycat, dlo, dhi, b, w, b3):
    return pl.pallas_call(
        _m3_body,
        grid=(GB,),
        in_specs=[
            pl.BlockSpec((BM, DH), lambda i: (i, 0)),
            pl.BlockSpec((BM, DH), lambda i: (GB + i, 0)),
            pl.BlockSpec((BM, DH), lambda i: (i, 0)),
            pl.BlockSpec((BM, DH), lambda i: (GB + i, 0)),
            pl.BlockSpec((BM, 1), lambda i: (i, 0)),
            pl.BlockSpec((BM, 1), lambda i: (i, 0)),
            pl.BlockSpec((1, D), lambda i: (0, 0)),
            pl.BlockSpec((D, D), lambda i: (0, 0)),
            pl.BlockSpec((1, D), lambda i: (0, 0)),
        ],
        out_specs=pl.BlockSpec((BM, D), lambda i: (i, 0)),
        out_shape=jax.ShapeDtypeStruct((NP, D), jnp.float32),
    )(acc, acc, ycat, ycat, dlo, dhi, b, w, b3)


def _first(x):
    return x[0] if isinstance(x, (tuple, list)) else x


def kernel(x, edge_index, edge_weight, W1, b1, W2, b2, W3, b3):
    row = edge_index[0].astype(jnp.int32)
    col = edge_index[1].astype(jnp.int32)
    ew = edge_weight.astype(jnp.float32)

    pad = EPAD - E
    row_p = jnp.concatenate([row, jnp.zeros((pad,), jnp.int32)]).reshape(ECH, K)
    col_p = jnp.concatenate([col, jnp.full((pad,), N, jnp.int32)]).reshape(ECH, K)
    ew_p = jnp.concatenate([ew, jnp.zeros((pad,), jnp.float32)]).reshape(ECH, K)
    x_p = jnp.pad(x, ((0, NP - N), (0, 0)))
    z2d = jnp.zeros((RPT, DH), jnp.float32)
    z1d = jnp.zeros((RPT,), jnp.float32)
    w1h = W1.astype(jnp.float32).reshape(D, NC, DH).transpose(1, 0, 2)
    w2h = W2.astype(jnp.float32).reshape(D, NC, DH).transpose(1, 0, 2)

    deg2 = _first(_sc_deg()(col_p, ew_p, z1d))
    dlo = deg2[0][:, None]
    dhi = deg2[1][:, None]

    y1 = _m1(x_p, w1h, dlo, dhi)
    acc1 = _first(_sc_layer()(row_p, col_p, ew_p, y1, z2d))
    y2 = _m2(acc1, y1, dlo, dhi, b1.reshape(1, D), w2h)
    acc2 = _first(_sc_layer()(row_p, col_p, ew_p, y2, z2d))
    out = _m3(acc2, y2, dlo, dhi, b2.reshape(1, D), W3, b3.reshape(1, D))
    return out[:N]
